# baseline (speedup 1.0000x reference)
import functools

import jax
import jax.numpy as jnp
from jax import lax
from jax.experimental import pallas as pl
from jax.experimental.pallas import tpu as pltpu
from jax.experimental.pallas import tpu_sc as plsc

_N = 10000
_E = 320000
_D = 128

_NC = 2
_NS = 16
_K = 128
_WIN = 16
_NWIN = 10
_CPT = _WIN * _NWIN
_EROWS = _NS * _CPT
_EPAD = _EROWS * _K
_NP = 10112
_PAD_DST = 10008
_RPT = _NP // _NS

_MESH = plsc.VectorSubcoreMesh(core_axis_name="c", subcore_axis_name="s",
                               num_cores=_NC, num_subcores=_NS)


def _agg_body(with_count, y_hbm, src_hbm, dst_hbm, z_hbm, ones_hbm, out_hbm,
              acc, srcv, dstv, rows_a, rows_b, sem_a, sem_b):
    c = lax.axis_index("c")
    s = lax.axis_index("s")
    r0 = s * _RPT

    pltpu.sync_copy(z_hbm.at[pl.ds(r0, _RPT)], acc.at[pl.ds(r0, _RPT)])
    if with_count:
        pltpu.sync_copy(ones_hbm, rows_a)
    plsc.subcore_barrier()

    def gather(j, rows, sem):
        pltpu.async_copy(y_hbm.at[srcv.at[j]], rows, sem)

    def drain_scatter(j, rows, sem):
        pltpu.make_async_copy(y_hbm.at[srcv.at[j]], rows, sem).wait()
        pltpu.sync_copy(rows, acc.at[dstv.at[j]], add=True)

    tile_chunk0 = s * _CPT

    nwin_g = jnp.where(c == 0, _NWIN, 0)

    @pl.loop(0, nwin_g)
    def _(w):
        i0 = tile_chunk0 + w * _WIN
        pltpu.sync_copy(src_hbm.at[pl.ds(i0, _WIN)], srcv)
        pltpu.sync_copy(dst_hbm.at[pl.ds(i0, _WIN)], dstv)

        gather(0, rows_a, sem_a)

        @pl.loop(0, _WIN // 2 - 1)
        def _(j2):
            ja = 2 * j2
            gather(ja + 1, rows_b, sem_b)
            drain_scatter(ja, rows_a, sem_a)
            gather(ja + 2, rows_a, sem_a)
            drain_scatter(ja + 1, rows_b, sem_b)

        gather(_WIN - 1, rows_b, sem_b)
        drain_scatter(_WIN - 2, rows_a, sem_a)
        drain_scatter(_WIN - 1, rows_b, sem_b)

    if with_count:
        nwin_c = jnp.where(c == 1, _NWIN, 0)

        @pl.loop(0, nwin_c)
        def _(w):
            i0 = tile_chunk0 + w * _WIN
            pltpu.sync_copy(dst_hbm.at[pl.ds(i0, _WIN)], dstv)

            @pl.loop(0, _WIN)
            def _(j):
                pltpu.sync_copy(rows_a, acc.at[dstv.at[j]], add=True)

    plsc.subcore_barrier()
    pltpu.sync_copy(acc.at[pl.ds(r0, _RPT)],
                    out_hbm.at[c].at[pl.ds(r0, _RPT)])


def _make_agg(with_count):
    return pl.kernel(
        functools.partial(_agg_body, with_count),
        out_type=jax.ShapeDtypeStruct((_NC, _NP, _D), jnp.float32),
        mesh=_MESH,
        scratch_types=[
            pltpu.VMEM_SHARED((_NP, _D), jnp.float32),
            pltpu.VMEM((_WIN, _K), jnp.int32),
            pltpu.VMEM((_WIN, _K), jnp.int32),
            pltpu.VMEM((_K, _D), jnp.float32),
            pltpu.VMEM((_K, _D), jnp.float32),
            pltpu.SemaphoreType.DMA,
            pltpu.SemaphoreType.DMA,
        ],
    )


_agg_cnt = _make_agg(True)
_agg = _make_agg(False)


def _lin_body(x_ref, w_ref, b_ref, o_ref):
    o_ref[...] = (
        jnp.dot(x_ref[...], w_ref[...],
                preferred_element_type=jnp.float32,
                precision=lax.Precision.HIGHEST)
        + b_ref[...]
    )


def _lin(x, w_t, b):
    r = 1000
    return pl.pallas_call(
        _lin_body,
        grid=(_N // r,),
        in_specs=[
            pl.BlockSpec((r, _D), lambda i: (i, 0)),
            pl.BlockSpec((_D, _D), lambda i: (0, 0)),
            pl.BlockSpec((1, _D), lambda i: (0, 0)),
        ],
        out_specs=pl.BlockSpec((r, _D), lambda i: (i, 0)),
        out_shape=jax.ShapeDtypeStruct((_N, _D), jnp.float32),
    )(x, w_t, b)


def _combine_body(relu, s_ref, c_ref, xr_ref, w_ref, o_ref):
    cnt = c_ref[...][:, :1]
    mean = s_ref[...] / jnp.maximum(cnt, 1.0)
    out = (
        jnp.dot(mean, w_ref[...],
                preferred_element_type=jnp.float32,
                precision=lax.Precision.HIGHEST)
        + xr_ref[...]
    )
    if relu:
        out = jnp.maximum(out, 0.0)
    o_ref[...] = out


def _combine(ssum, cnt, xr, w_t, relu):
    r = 1000
    return pl.pallas_call(
        functools.partial(_combine_body, relu),
        grid=(_N // r,),
        in_specs=[
            pl.BlockSpec((r, _D), lambda i: (i, 0)),
            pl.BlockSpec((r, _D), lambda i: (i, 0)),
            pl.BlockSpec((r, _D), lambda i: (i, 0)),
            pl.BlockSpec((_D, _D), lambda i: (0, 0)),
        ],
        out_specs=pl.BlockSpec((r, _D), lambda i: (i, 0)),
        out_shape=jax.ShapeDtypeStruct((_N, _D), jnp.float32),
    )(ssum, cnt, xr, w_t)


def kernel(x, edge_index, W1_l, b1, W1_r, W2_l, b2, W2_r):
    npad = _EPAD - _E
    src = jnp.concatenate(
        [edge_index[0].astype(jnp.int32),
         jnp.zeros((npad,), jnp.int32)]).reshape(_EROWS, _K)
    dst = jnp.concatenate(
        [edge_index[1].astype(jnp.int32),
         jnp.full((npad,), _PAD_DST, jnp.int32)]).reshape(_EROWS, _K)
    zeros_acc = jnp.zeros((_NP, _D), jnp.float32)
    ones = jnp.ones((_K, _D), jnp.float32)

    a1 = _agg_cnt(x, src, dst, zeros_acc, ones)
    xr1 = _lin(x, W1_r.T, b1.reshape(1, _D))
    h = _combine(a1[0], a1[1], xr1, W1_l.T, relu=True)

    a2 = _agg(h, src, dst, zeros_acc, ones)
    xr2 = _lin(h, W2_r.T, b2.reshape(1, _D))
    out = _combine(a2[0], a1[1], xr2, W2_l.T, relu=False)
    return out


# speedup vs baseline: 2.3814x; 2.3814x over previous
import functools

import jax
import jax.numpy as jnp
from jax import lax
from jax.experimental import pallas as pl
from jax.experimental.pallas import tpu as pltpu
from jax.experimental.pallas import tpu_sc as plsc

_N = 10000
_E = 320000
_D = 128

_NC = 2
_NS = 16
_K = 128
_WIN = 16
_NWIN = 10
_CPT = _WIN * _NWIN
_EROWS = _NS * _CPT
_EPAD = _EROWS * _K
_NP = 10112
_PAD_DST = 10008
_RPT = _NP // _NS

_MESH = plsc.VectorSubcoreMesh(core_axis_name="c", subcore_axis_name="s",
                               num_cores=_NC, num_subcores=_NS)


def _agg_body(with_count, y_hbm, src_hbm, dst_hbm, z_hbm, ones_hbm, out_hbm,
              acc, srcv, dstv, rows_a, rows_b, sem_a, sem_b):
    c = lax.axis_index("c")
    s = lax.axis_index("s")
    r0 = s * _RPT

    pltpu.sync_copy(z_hbm.at[pl.ds(r0, _RPT)], acc.at[pl.ds(r0, _RPT)])
    if with_count:
        pltpu.sync_copy(ones_hbm, rows_a)
    plsc.subcore_barrier()

    def gather(j, rows, sem):
        pltpu.async_copy(y_hbm.at[srcv.at[j]], rows, sem)

    def drain_scatter(j, rows, sem):
        pltpu.make_async_copy(y_hbm.at[srcv.at[j]], rows, sem).wait()
        pltpu.sync_copy(rows, acc.at[dstv.at[j]], add=True)

    tile_chunk0 = s * _CPT

    nwin_g = jnp.where(c == 0, _NWIN, 0)

    @pl.loop(0, nwin_g)
    def _(w):
        i0 = tile_chunk0 + w * _WIN
        pltpu.sync_copy(src_hbm.at[pl.ds(i0, _WIN)], srcv)
        pltpu.sync_copy(dst_hbm.at[pl.ds(i0, _WIN)], dstv)

        gather(0, rows_a, sem_a)

        @pl.loop(0, _WIN // 2 - 1)
        def _(j2):
            ja = 2 * j2
            gather(ja + 1, rows_b, sem_b)
            drain_scatter(ja, rows_a, sem_a)
            gather(ja + 2, rows_a, sem_a)
            drain_scatter(ja + 1, rows_b, sem_b)

        gather(_WIN - 1, rows_b, sem_b)
        drain_scatter(_WIN - 2, rows_a, sem_a)
        drain_scatter(_WIN - 1, rows_b, sem_b)

    if with_count:
        nwin_c = jnp.where(c == 1, _NWIN, 0)

        @pl.loop(0, nwin_c)
        def _(w):
            i0 = tile_chunk0 + w * _WIN
            pltpu.sync_copy(dst_hbm.at[pl.ds(i0, _WIN)], dstv)

            @pl.loop(0, _WIN)
            def _(j):
                pltpu.sync_copy(rows_a, acc.at[dstv.at[j]], add=True)

    plsc.subcore_barrier()
    pltpu.sync_copy(acc.at[pl.ds(r0, _RPT)],
                    out_hbm.at[c].at[pl.ds(r0, _RPT)])


def _make_agg(with_count):
    return pl.kernel(
        functools.partial(_agg_body, with_count),
        out_type=jax.ShapeDtypeStruct((_NC, _NP, _D), jnp.float32),
        mesh=_MESH,
        scratch_types=[
            pltpu.VMEM_SHARED((_NP, _D), jnp.float32),
            pltpu.VMEM((_WIN, _K), jnp.int32),
            pltpu.VMEM((_WIN, _K), jnp.int32),
            pltpu.VMEM((_K, _D), jnp.float32),
            pltpu.VMEM((_K, _D), jnp.float32),
            pltpu.SemaphoreType.DMA,
            pltpu.SemaphoreType.DMA,
        ],
    )


_agg_cnt = _make_agg(True)
_agg = _make_agg(False)


def _lin_body(x_ref, w_ref, b_ref, o_ref):
    o_ref[...] = (
        jnp.dot(x_ref[...], w_ref[...],
                preferred_element_type=jnp.float32,
                precision=lax.Precision.HIGHEST)
        + b_ref[...]
    )


def _lin(x, w_t, b):
    r = 1000
    return pl.pallas_call(
        _lin_body,
        grid=(_N // r,),
        in_specs=[
            pl.BlockSpec((r, _D), lambda i: (i, 0)),
            pl.BlockSpec((_D, _D), lambda i: (0, 0)),
            pl.BlockSpec((1, _D), lambda i: (0, 0)),
        ],
        out_specs=pl.BlockSpec((r, _D), lambda i: (i, 0)),
        out_shape=jax.ShapeDtypeStruct((_N, _D), jnp.float32),
    )(x, w_t, b)


def _combine_body(relu, s_ref, c_ref, xr_ref, w_ref, o_ref):
    cnt = c_ref[...][:, :1]
    mean = s_ref[...] / jnp.maximum(cnt, 1.0)
    out = (
        jnp.dot(mean, w_ref[...],
                preferred_element_type=jnp.float32,
                precision=lax.Precision.HIGHEST)
        + xr_ref[...]
    )
    if relu:
        out = jnp.maximum(out, 0.0)
    o_ref[...] = out


def _combine(ssum, cnt, xr, w_t, relu):
    r = 1000
    return pl.pallas_call(
        functools.partial(_combine_body, relu),
        grid=(_N // r,),
        in_specs=[
            pl.BlockSpec((r, _D), lambda i: (i, 0)),
            pl.BlockSpec((r, _D), lambda i: (i, 0)),
            pl.BlockSpec((r, _D), lambda i: (i, 0)),
            pl.BlockSpec((_D, _D), lambda i: (0, 0)),
        ],
        out_specs=pl.BlockSpec((r, _D), lambda i: (i, 0)),
        out_shape=jax.ShapeDtypeStruct((_N, _D), jnp.float32),
    )(ssum, cnt, xr, w_t)


def kernel(x, edge_index, W1_l, b1, W1_r, W2_l, b2, W2_r):
    npad = _EPAD - _E
    pad_iota = jax.lax.iota(jnp.int32, npad)
    src = jnp.concatenate(
        [edge_index[0].astype(jnp.int32),
         pad_iota % _N]).reshape(_EROWS, _K)
    dst = jnp.concatenate(
        [edge_index[1].astype(jnp.int32),
         _PAD_DST + pad_iota % (_NP - _PAD_DST)]).reshape(_EROWS, _K)
    zeros_acc = jnp.zeros((_NP, _D), jnp.float32)
    ones = jnp.ones((_K, _D), jnp.float32)

    a1 = _agg_cnt(x, src, dst, zeros_acc, ones)
    xr1 = _lin(x, W1_r.T, b1.reshape(1, _D))
    h = _combine(a1[0], a1[1], xr1, W1_l.T, relu=True)

    a2 = _agg(h, src, dst, zeros_acc, ones)
    xr2 = _lin(h, W2_r.T, b2.reshape(1, _D))
    out = _combine(a2[0], a1[1], xr2, W2_l.T, relu=False)
    return out


# speedup vs baseline: 2.9263x; 1.2288x over previous
import functools

import jax
import jax.numpy as jnp
from jax import lax
from jax.experimental import pallas as pl
from jax.experimental.pallas import tpu as pltpu
from jax.experimental.pallas import tpu_sc as plsc

_N = 10000
_E = 320000
_D = 128

_NC = 2
_NS = 16
_K = 128
_WIN = 16
_NWIN = 10
_CPT = _WIN * _NWIN
_EROWS = _NS * _CPT
_EPAD = _EROWS * _K
_NP = 10112
_PAD_DST = 10008
_RPT = _NP // _NS

_MESH = plsc.VectorSubcoreMesh(core_axis_name="c", subcore_axis_name="s",
                               num_cores=_NC, num_subcores=_NS)


def _agg_body(with_count, y_hbm, src_hbm, dst_hbm, z_hbm, ones_hbm, out_hbm,
              acc, srcv, dstv, rows_a, rows_b, sem_a, sem_b):
    c = lax.axis_index("c")
    s = lax.axis_index("s")
    r0 = s * _RPT

    pltpu.sync_copy(z_hbm.at[pl.ds(r0, _RPT)], acc.at[pl.ds(r0, _RPT)])
    if with_count:
        pltpu.sync_copy(ones_hbm, rows_a)
    plsc.subcore_barrier()

    def gather(j, rows, sem):
        pltpu.async_copy(y_hbm.at[srcv.at[j]], rows, sem)

    def drain_scatter(j, rows, sem):
        pltpu.make_async_copy(y_hbm.at[srcv.at[j]], rows, sem).wait()
        pltpu.sync_copy(rows, acc.at[dstv.at[j]], add=True)

    tile_chunk0 = s * _CPT

    if with_count:
        nwin_g = jnp.where(c == 0, _NWIN, 0)
        w_base = 0 * c
    else:
        nwin_g = _NWIN // 2
        w_base = jnp.where(c == 0, 0, _NWIN // 2)

    @pl.loop(0, nwin_g)
    def _(w):
        i0 = tile_chunk0 + (w_base + w) * _WIN
        pltpu.sync_copy(src_hbm.at[pl.ds(i0, _WIN)], srcv)
        pltpu.sync_copy(dst_hbm.at[pl.ds(i0, _WIN)], dstv)

        gather(0, rows_a, sem_a)

        @pl.loop(0, _WIN // 2 - 1)
        def _(j2):
            ja = 2 * j2
            gather(ja + 1, rows_b, sem_b)
            drain_scatter(ja, rows_a, sem_a)
            gather(ja + 2, rows_a, sem_a)
            drain_scatter(ja + 1, rows_b, sem_b)

        gather(_WIN - 1, rows_b, sem_b)
        drain_scatter(_WIN - 2, rows_a, sem_a)
        drain_scatter(_WIN - 1, rows_b, sem_b)

    if with_count:
        nwin_c = jnp.where(c == 1, _NWIN, 0)

        @pl.loop(0, nwin_c)
        def _(w):
            i0 = tile_chunk0 + w * _WIN
            pltpu.sync_copy(dst_hbm.at[pl.ds(i0, _WIN)], dstv)

            @pl.loop(0, _WIN)
            def _(j):
                pltpu.sync_copy(rows_a, acc.at[dstv.at[j]], add=True)

    plsc.subcore_barrier()
    pltpu.sync_copy(acc.at[pl.ds(r0, _RPT)],
                    out_hbm.at[c].at[pl.ds(r0, _RPT)])


def _make_agg(with_count):
    return pl.kernel(
        functools.partial(_agg_body, with_count),
        out_type=jax.ShapeDtypeStruct((_NC, _NP, _D), jnp.float32),
        mesh=_MESH,
        scratch_types=[
            pltpu.VMEM_SHARED((_NP, _D), jnp.float32),
            pltpu.VMEM((_WIN, _K), jnp.int32),
            pltpu.VMEM((_WIN, _K), jnp.int32),
            pltpu.VMEM((_K, _D), jnp.float32),
            pltpu.VMEM((_K, _D), jnp.float32),
            pltpu.SemaphoreType.DMA,
            pltpu.SemaphoreType.DMA,
        ],
    )


_agg_cnt = _make_agg(True)
_agg = _make_agg(False)


def _lin_body(x_ref, w_ref, b_ref, o_ref):
    o_ref[...] = (
        jnp.dot(x_ref[...], w_ref[...],
                preferred_element_type=jnp.float32,
                precision=lax.Precision.HIGHEST)
        + b_ref[...]
    )


def _lin(x, w_t, b):
    r = 1000
    return pl.pallas_call(
        _lin_body,
        grid=(_N // r,),
        in_specs=[
            pl.BlockSpec((r, _D), lambda i: (i, 0)),
            pl.BlockSpec((_D, _D), lambda i: (0, 0)),
            pl.BlockSpec((1, _D), lambda i: (0, 0)),
        ],
        out_specs=pl.BlockSpec((r, _D), lambda i: (i, 0)),
        out_shape=jax.ShapeDtypeStruct((_N, _D), jnp.float32),
    )(x, w_t, b)


def _combine_body(relu, s_ref, s1_ref, c_ref, xr_ref, w_ref, o_ref):
    ssum = s_ref[...] if s1_ref is None else s_ref[...] + s1_ref[...]
    cnt = c_ref[...][:, :1]
    mean = ssum / jnp.maximum(cnt, 1.0)
    out = (
        jnp.dot(mean, w_ref[...],
                preferred_element_type=jnp.float32,
                precision=lax.Precision.HIGHEST)
        + xr_ref[...]
    )
    if relu:
        out = jnp.maximum(out, 0.0)
    o_ref[...] = out


def _combine(s0, s1, cnt, xr, w_t, relu):
    r = 1000
    two = s1 is not None
    def body(*refs):
        if two:
            _combine_body(relu, *refs)
        else:
            sref, cref, xref, wref, oref = refs
            _combine_body(relu, sref, None, cref, xref, wref, oref)
    specs = [pl.BlockSpec((r, _D), lambda i: (i, 0))]
    if two:
        specs.append(pl.BlockSpec((r, _D), lambda i: (i, 0)))
    specs += [
        pl.BlockSpec((r, _D), lambda i: (i, 0)),
        pl.BlockSpec((r, _D), lambda i: (i, 0)),
        pl.BlockSpec((_D, _D), lambda i: (0, 0)),
    ]
    args = ([s0, s1] if two else [s0]) + [cnt, xr, w_t]
    return pl.pallas_call(
        body,
        grid=(_N // r,),
        in_specs=specs,
        out_specs=pl.BlockSpec((r, _D), lambda i: (i, 0)),
        out_shape=jax.ShapeDtypeStruct((_N, _D), jnp.float32),
    )(*args)


def kernel(x, edge_index, W1_l, b1, W1_r, W2_l, b2, W2_r):
    npad = _EPAD - _E
    pad_iota = jax.lax.iota(jnp.int32, npad)
    src = jnp.concatenate(
        [edge_index[0].astype(jnp.int32),
         pad_iota % _N]).reshape(_EROWS, _K)
    dst = jnp.concatenate(
        [edge_index[1].astype(jnp.int32),
         _PAD_DST + pad_iota % (_NP - _PAD_DST)]).reshape(_EROWS, _K)
    zeros_acc = jnp.zeros((_NP, _D), jnp.float32)
    ones = jnp.ones((_K, _D), jnp.float32)

    a1 = _agg_cnt(x, src, dst, zeros_acc, ones)
    xr1 = _lin(x, W1_r.T, b1.reshape(1, _D))
    h = _combine(a1[0], None, a1[1], xr1, W1_l.T, relu=True)

    a2 = _agg(h, src, dst, zeros_acc, ones)
    xr2 = _lin(h, W2_r.T, b2.reshape(1, _D))
    out = _combine(a2[0], a2[1], a1[1], xr2, W2_l.T, relu=False)
    return out
